# baseline (speedup 1.0000x reference)
import functools

import jax
import jax.numpy as jnp
from jax import lax
from jax.experimental import pallas as pl
from jax.experimental.pallas import tpu as pltpu
from jax.experimental.pallas import tpu_sc as plsc

F = 26
V = 100000
E = 32
B = 16384
ROWS = B * F

NC = 2
NS = 16
NW = NC * NS
RPW = ROWS // NW

CHUNK = 1664
NCHUNKS = RPW // CHUNK
GSTEP = 128
NG = CHUNK // GSTEP

_mesh = plsc.VectorSubcoreMesh(core_axis_name="c", subcore_axis_name="s")


@functools.partial(
    pl.kernel,
    out_type=jax.ShapeDtypeStruct((ROWS, E), jnp.float32),
    mesh=_mesh,
    compiler_params=pltpu.CompilerParams(use_tc_tiling_on_sc=False),
    scratch_types=[
        pltpu.VMEM((RPW,), jnp.int32),
        pltpu.VMEM((CHUNK,), jnp.int32),
        pltpu.VMEM((2, CHUNK, E), jnp.float32),
        pltpu.SemaphoreType.DMA,
        pltpu.SemaphoreType.DMA,
    ],
)
def _sc_gather(table, xflat, offs, out, idx_v, off_v, rows_v, gsem, ssem):
    w = lax.axis_index("s") * NC + lax.axis_index("c")
    base = w * RPW

    pltpu.sync_copy(xflat.at[pl.ds(base, RPW)], idx_v)
    pltpu.sync_copy(offs, off_v)

    def add_offs(i, carry):
        p = pl.multiple_of(i * 16, 16)
        q = pl.multiple_of(lax.rem(i, CHUNK // 16) * 16, 16)
        idx_v[pl.ds(p, 16)] = idx_v[pl.ds(p, 16)] + off_v[pl.ds(q, 16)]
        return carry

    lax.fori_loop(0, RPW // 16, add_offs, 0)

    store_handles = []
    for c in range(NCHUNKS):
        buf = c % 2
        if c >= 2:
            store_handles[c - 2].wait()
        gather_handles = []
        for g in range(NG):
            s = c * CHUNK + g * GSTEP
            h = pltpu.async_copy(
                table.at[idx_v.at[pl.ds(s, GSTEP)]],
                rows_v.at[buf, pl.ds(g * GSTEP, GSTEP)],
                gsem,
            )
            gather_handles.append(h)
        for h in gather_handles:
            h.wait()
        store_handles.append(
            pltpu.async_copy(
                rows_v.at[buf],
                out.at[pl.ds(base + c * CHUNK, CHUNK)],
                ssem,
            )
        )
    store_handles[-2].wait()
    store_handles[-1].wait()


def kernel(x, embedding):
    xflat = x.reshape(-1).astype(jnp.int32)
    table = embedding.reshape(F * V, E)
    offs = (jnp.arange(CHUNK, dtype=jnp.int32) % F) * V
    out = _sc_gather(table, xflat, offs)
    return out.reshape(B, F, E)


# speedup vs baseline: 2.9293x; 2.9293x over previous
import functools

import jax
import jax.numpy as jnp
from jax import lax
from jax.experimental import pallas as pl
from jax.experimental.pallas import tpu as pltpu
from jax.experimental.pallas import tpu_sc as plsc

F = 26
V = 100000
E = 32
B = 16384

NC = 2
NS = 16
NW = NC * NS

CHUNK = 2048
NCHUNKS = B // CHUNK

_mesh = plsc.VectorSubcoreMesh(core_axis_name="c", subcore_axis_name="s")


@functools.partial(
    pl.kernel,
    out_type=jax.ShapeDtypeStruct((F * E * B,), jnp.float32),
    mesh=_mesh,
    compiler_params=pltpu.CompilerParams(
        use_tc_tiling_on_sc=True, needs_layout_passes=False
    ),
    scratch_types=[
        pltpu.VMEM((V,), jnp.float32),
        pltpu.VMEM((B,), jnp.int32),
        pltpu.VMEM((2, CHUNK), jnp.float32),
        pltpu.SemaphoreType.DMA,
    ],
)
def _sc_lookup(table_t, xcols, out, row_v, idx_v, out_v, ssem):
    e = lax.axis_index("s") * NC + lax.axis_index("c")

    def body_f(f, carry):
        pltpu.sync_copy(table_t.at[f, e, :], row_v)
        pltpu.sync_copy(xcols.at[pl.ds(f * B, B)], idx_v)
        obase = (f * E + e) * B

        def body_c(c, carry2):
            buf = lax.rem(c, 2)

            def g16(i, carry3):
                s = pl.multiple_of(c * CHUNK + i * 16, 16)
                idx = idx_v[pl.ds(s, 16)]
                out_v[buf, pl.ds(pl.multiple_of(i * 16, 16), 16)] = (
                    plsc.load_gather(row_v, [idx])
                )
                return carry3

            lax.fori_loop(0, CHUNK // 16, g16, 0)
            pltpu.sync_copy(
                out_v.at[buf], out.at[pl.ds(obase + c * CHUNK, CHUNK)]
            )
            return carry2

        lax.fori_loop(0, NCHUNKS, body_c, 0)
        return carry

    lax.fori_loop(0, F, body_f, 0)


def kernel(x, embedding):
    table_t = jnp.transpose(embedding, (0, 2, 1))
    xcols = jnp.transpose(x.astype(jnp.int32)).reshape(-1)
    out = _sc_lookup(table_t, xcols)
    return jnp.transpose(out.reshape(F, E, B), (2, 0, 1))


# speedup vs baseline: 3.1779x; 1.0849x over previous
import functools

import jax
import jax.numpy as jnp
from jax import lax
from jax.experimental import pallas as pl
from jax.experimental.pallas import tpu as pltpu
from jax.experimental.pallas import tpu_sc as plsc

F = 26
V = 100000
E = 32
B = 16384

NC = 2
NS = 16
NW = NC * NS

CHUNK = 2048
NCHUNKS = B // CHUNK

_mesh = plsc.VectorSubcoreMesh(core_axis_name="c", subcore_axis_name="s")


@functools.partial(
    pl.kernel,
    out_type=jax.ShapeDtypeStruct((F * E * B,), jnp.float32),
    mesh=_mesh,
    compiler_params=pltpu.CompilerParams(
        use_tc_tiling_on_sc=True, needs_layout_passes=False
    ),
    scratch_types=[
        pltpu.VMEM((V,), jnp.float32),
        pltpu.VMEM((2, CHUNK), jnp.int32),
        pltpu.VMEM((2, CHUNK), jnp.float32),
        pltpu.SemaphoreType.DMA,
        pltpu.SemaphoreType.DMA,
        pltpu.SemaphoreType.DMA,
    ],
)
def _sc_lookup(table_t, xcols, out, row_v, idx_v, out_v, rsem, isem, ssem):
    e = lax.axis_index("s") * NC + lax.axis_index("c")

    def idx_copy(f, c, buf):
        return pltpu.async_copy(
            xcols.at[pl.ds(f * B + c * CHUNK, CHUNK)], idx_v.at[buf], isem
        )

    def body_f(f, carry):
        row_cp = pltpu.async_copy(table_t.at[f, e, :], row_v, rsem)
        idx_copy(f, 0, 0).wait()
        row_cp.wait()
        obase = (f * E + e) * B

        def body_c(c, carry2):
            buf = lax.rem(c, 2)
            nc_ = c + 1
            nf = lax.select(nc_ == NCHUNKS, f + 1, f)
            nc_ = lax.rem(nc_, NCHUNKS)
            pred = (f < F - 1) | (nc_ > 0)

            @pl.when(pred)
            def _():
                idx_copy(nf, nc_, 1 - buf)

            @pl.when(c >= 2)
            def _():
                pltpu.make_async_copy(
                    out_v.at[buf], out.at[pl.ds(obase, CHUNK)], ssem
                ).wait()

            def g16(i, carry3):
                s = pl.multiple_of(i * 16, 16)
                out_v[buf, pl.ds(s, 16)] = plsc.load_gather(
                    row_v, [idx_v[buf, pl.ds(s, 16)]]
                )
                return carry3

            lax.fori_loop(0, CHUNK // 16, g16, 0, unroll=8)
            pltpu.async_copy(
                out_v.at[buf], out.at[pl.ds(obase + c * CHUNK, CHUNK)], ssem
            )

            @pl.when(pred)
            def _():
                pltpu.make_async_copy(
                    xcols.at[pl.ds(0, CHUNK)], idx_v.at[1 - buf], isem
                ).wait()
            return carry2

        lax.fori_loop(0, NCHUNKS, body_c, 0)
        pltpu.make_async_copy(
            out_v.at[0], out.at[pl.ds(obase, CHUNK)], ssem
        ).wait()
        pltpu.make_async_copy(
            out_v.at[1], out.at[pl.ds(obase, CHUNK)], ssem
        ).wait()
        return carry

    lax.fori_loop(0, F, body_f, 0)


def kernel(x, embedding):
    table_t = jnp.transpose(embedding, (0, 2, 1))
    xcols = jnp.transpose(x.astype(jnp.int32)).reshape(-1)
    out = _sc_lookup(table_t, xcols)
    return jnp.transpose(out.reshape(F, E, B), (2, 0, 1))


# speedup vs baseline: 3.5991x; 1.1326x over previous
import functools

import jax
import jax.numpy as jnp
from jax import lax
from jax.experimental import pallas as pl
from jax.experimental.pallas import tpu as pltpu
from jax.experimental.pallas import tpu_sc as plsc

F = 26
V = 100000
E = 32
B = 16384

NC = 2
NS = 16
NW = NC * NS

CHUNK = 2048
NCHUNKS = B // CHUNK

_mesh = plsc.VectorSubcoreMesh(core_axis_name="c", subcore_axis_name="s")


@functools.partial(
    pl.kernel,
    out_type=jax.ShapeDtypeStruct((F, E, B), jnp.float32),
    mesh=_mesh,
    compiler_params=pltpu.CompilerParams(
        use_tc_tiling_on_sc=True, needs_layout_passes=False
    ),
    scratch_types=[
        pltpu.VMEM((V,), jnp.float32),
        pltpu.VMEM((2, CHUNK), jnp.int32),
        pltpu.VMEM((B,), jnp.float32),
        pltpu.SemaphoreType.DMA,
        pltpu.SemaphoreType.DMA,
        pltpu.SemaphoreType.DMA,
    ],
)
def _sc_lookup(table_t, xcols, out, row_v, idx_v, out_v, rsem, isem, ssem):
    e = lax.axis_index("s") * NC + lax.axis_index("c")

    def idx_copy(f, c, buf):
        return pltpu.async_copy(
            xcols.at[pl.ds(f * B + c * CHUNK, CHUNK)], idx_v.at[buf], isem
        )

    def body_f(f, carry):
        row_cp = pltpu.async_copy(table_t.at[f, e, :], row_v, rsem)
        idx_copy(f, 0, 0).wait()
        row_cp.wait()

        @pl.when(f > 0)
        def _():
            pltpu.make_async_copy(out_v, out.at[f, e, :], ssem).wait()

        def body_c(c, carry2):
            buf = lax.rem(c, 2)
            nc_ = c + 1
            nf = lax.select(nc_ == NCHUNKS, f + 1, f)
            nc_ = lax.rem(nc_, NCHUNKS)
            pred = (f < F - 1) | (nc_ > 0)

            @pl.when(pred)
            def _():
                idx_copy(nf, nc_, 1 - buf)

            def g16(i, carry3):
                s = pl.multiple_of(i * 16, 16)
                out_v[pl.ds(c * CHUNK + s, 16)] = plsc.load_gather(
                    row_v, [idx_v[buf, pl.ds(s, 16)]]
                )
                return carry3

            lax.fori_loop(0, CHUNK // 16, g16, 0, unroll=8)

            @pl.when(pred)
            def _():
                pltpu.make_async_copy(
                    xcols.at[pl.ds(0, CHUNK)], idx_v.at[1 - buf], isem
                ).wait()
            return carry2

        lax.fori_loop(0, NCHUNKS, body_c, 0)
        pltpu.async_copy(out_v, out.at[f, e, :], ssem)
        return carry

    lax.fori_loop(0, F, body_f, 0)
    pltpu.make_async_copy(out_v, out.at[F - 1, e, :], ssem).wait()


def kernel(x, embedding):
    table_t = jnp.transpose(embedding, (0, 2, 1))
    xcols = jnp.transpose(x.astype(jnp.int32)).reshape(-1)
    out = _sc_lookup(table_t, xcols)
    return jnp.transpose(out, (2, 0, 1))


# speedup vs baseline: 3.6196x; 1.0057x over previous
import functools

import jax
import jax.numpy as jnp
from jax import lax
from jax.experimental import pallas as pl
from jax.experimental.pallas import tpu as pltpu
from jax.experimental.pallas import tpu_sc as plsc

F = 26
V = 100000
E = 32
B = 16384

NC = 2
NS = 16
NW = NC * NS

CHUNK = 2048
NCHUNKS = B // CHUNK

_mesh = plsc.VectorSubcoreMesh(core_axis_name="c", subcore_axis_name="s")


@functools.partial(
    pl.kernel,
    out_type=jax.ShapeDtypeStruct((F, E, B), jnp.float32),
    mesh=_mesh,
    compiler_params=pltpu.CompilerParams(
        use_tc_tiling_on_sc=True, needs_layout_passes=False
    ),
    scratch_types=[
        pltpu.VMEM((V,), jnp.float32),
        pltpu.VMEM((2, CHUNK), jnp.int32),
        pltpu.VMEM((B,), jnp.float32),
        pltpu.SemaphoreType.DMA,
        pltpu.SemaphoreType.DMA,
        pltpu.SemaphoreType.DMA,
    ],
)
def _sc_lookup(table_t, xcols, out, row_v, idx_v, out_v, rsem, isem, ssem):
    e = lax.axis_index("s") * NC + lax.axis_index("c")

    def idx_copy(f, c, buf):
        return pltpu.async_copy(
            xcols.at[pl.ds(f * B + c * CHUNK, CHUNK)], idx_v.at[buf], isem
        )

    def body_f(f, carry):
        row_cp = pltpu.async_copy(table_t.at[f, e, :], row_v, rsem)
        idx_copy(f, 0, 0).wait()
        row_cp.wait()

        @pl.when(f > 0)
        def _():
            pltpu.make_async_copy(out_v, out.at[f, e, :], ssem).wait()

        def body_c(c, carry2):
            buf = lax.rem(c, 2)
            nc_ = c + 1
            nf = lax.select(nc_ == NCHUNKS, f + 1, f)
            nc_ = lax.rem(nc_, NCHUNKS)
            pred = (f < F - 1) | (nc_ > 0)

            @pl.when(pred)
            def _():
                idx_copy(nf, nc_, 1 - buf)

            def g16(i, carry3):
                s = pl.multiple_of(i * 16, 16)
                out_v[pl.ds(c * CHUNK + s, 16)] = plsc.load_gather(
                    row_v, [idx_v[buf, pl.ds(s, 16)]]
                )
                return carry3

            lax.fori_loop(0, CHUNK // 16, g16, 0, unroll=16)

            @pl.when(pred)
            def _():
                pltpu.make_async_copy(
                    xcols.at[pl.ds(0, CHUNK)], idx_v.at[1 - buf], isem
                ).wait()
            return carry2

        lax.fori_loop(0, NCHUNKS, body_c, 0)
        pltpu.async_copy(out_v, out.at[f, e, :], ssem)
        return carry

    lax.fori_loop(0, F, body_f, 0)
    pltpu.make_async_copy(out_v, out.at[F - 1, e, :], ssem).wait()


def kernel(x, embedding):
    table_t = jnp.transpose(embedding, (0, 2, 1))
    xcols = jnp.transpose(x.astype(jnp.int32)).reshape(-1)
    out = _sc_lookup(table_t, xcols)
    return jnp.transpose(out, (2, 0, 1))


# speedup vs baseline: 3.6345x; 1.0041x over previous
import functools

import jax
import jax.numpy as jnp
from jax import lax
from jax.experimental import pallas as pl
from jax.experimental.pallas import tpu as pltpu
from jax.experimental.pallas import tpu_sc as plsc

F = 26
V = 100000
E = 32
B = 16384

NC = 2
NS = 16
NW = NC * NS

CHUNK = 2048
NCHUNKS = B // CHUNK

_mesh = plsc.VectorSubcoreMesh(core_axis_name="c", subcore_axis_name="s")


@functools.partial(
    pl.kernel,
    out_type=jax.ShapeDtypeStruct((F, E, B), jnp.float32),
    mesh=_mesh,
    compiler_params=pltpu.CompilerParams(
        use_tc_tiling_on_sc=True, needs_layout_passes=False
    ),
    scratch_types=[
        pltpu.VMEM((V,), jnp.float32),
        pltpu.VMEM((2, CHUNK), jnp.int32),
        pltpu.VMEM((B,), jnp.float32),
        pltpu.VMEM_SHARED((2, B), jnp.int32),
        pltpu.SemaphoreType.DMA,
        pltpu.SemaphoreType.DMA,
        pltpu.SemaphoreType.DMA,
        pltpu.SemaphoreType.DMA,
    ],
)
def _sc_lookup(table_t, x_t, out, row_v, idx_v, out_v, xs_s, rsem, isem, ssem, lsem):
    e = lax.axis_index("s") * NC + lax.axis_index("c")
    lead = lax.axis_index("s") == 0

    @pl.when(lead)
    def _():
        pltpu.sync_copy(x_t.at[0, :], xs_s.at[0])

    plsc.subcore_barrier()

    def idx_copy(slot, c, buf):
        return pltpu.async_copy(
            xs_s.at[slot, pl.ds(c * CHUNK, CHUNK)], idx_v.at[buf], isem
        )

    def body_f(f, carry):
        slot = lax.rem(f, 2)

        @pl.when(lead & (f < F - 1))
        def _():
            pltpu.async_copy(x_t.at[f + 1, :], xs_s.at[1 - slot], lsem)

        row_cp = pltpu.async_copy(table_t.at[f, e, :], row_v, rsem)
        idx_copy(slot, 0, 0).wait()
        row_cp.wait()

        @pl.when(f > 0)
        def _():
            pltpu.make_async_copy(out_v, out.at[f, e, :], ssem).wait()

        def body_c(c, carry2):
            buf = lax.rem(c, 2)
            @pl.when(c < NCHUNKS - 1)
            def _():
                idx_copy(slot, c + 1, 1 - buf)

            def g16(i, carry3):
                s = pl.multiple_of(i * 16, 16)
                out_v[pl.ds(c * CHUNK + s, 16)] = plsc.load_gather(
                    row_v, [idx_v[buf, pl.ds(s, 16)]]
                )
                return carry3

            lax.fori_loop(0, CHUNK // 16, g16, 0, unroll=16)

            @pl.when(c < NCHUNKS - 1)
            def _():
                pltpu.make_async_copy(
                    xs_s.at[0, pl.ds(0, CHUNK)], idx_v.at[1 - buf], isem
                ).wait()
            return carry2

        lax.fori_loop(0, NCHUNKS, body_c, 0)
        pltpu.async_copy(out_v, out.at[f, e, :], ssem)

        @pl.when(lead & (f < F - 1))
        def _():
            pltpu.make_async_copy(
                x_t.at[0, :], xs_s.at[1 - slot], lsem
            ).wait()

        @pl.when(f < F - 1)
        def _():
            plsc.subcore_barrier()
        return carry

    lax.fori_loop(0, F, body_f, 0)
    pltpu.make_async_copy(out_v, out.at[F - 1, e, :], ssem).wait()


def kernel(x, embedding):
    table_t = jnp.transpose(embedding, (0, 2, 1))
    x_t = jnp.transpose(x.astype(jnp.int32))
    out = _sc_lookup(table_t, x_t)
    return jnp.transpose(out, (2, 0, 1))


# speedup vs baseline: 4.8192x; 1.3260x over previous
import functools

import jax
import jax.numpy as jnp
from jax import lax
from jax.experimental import pallas as pl
from jax.experimental.pallas import tpu as pltpu
from jax.experimental.pallas import tpu_sc as plsc

F = 26
V = 100000
E = 32
B = 16384

NC = 2
NS = 16
NW = NC * NS

_mesh = plsc.VectorSubcoreMesh(core_axis_name="c", subcore_axis_name="s")


@functools.partial(
    pl.kernel,
    out_type=jax.ShapeDtypeStruct((F, E, B), jnp.float32),
    mesh=_mesh,
    compiler_params=pltpu.CompilerParams(
        use_tc_tiling_on_sc=True, needs_layout_passes=False
    ),
    scratch_types=[
        pltpu.VMEM((V,), jnp.float32),
        pltpu.VMEM((B,), jnp.float32),
        pltpu.SemaphoreType.DMA,
        pltpu.SemaphoreType.DMA,
        pltpu.SemaphoreType.DMA,
    ],
)
def _sc_lookup(table_t, x_f, out, row_v, io_v, rsem, isem, ssem):
    e = lax.axis_index("s") * NC + lax.axis_index("c")

    def body_f(f, carry):
        row_cp = pltpu.async_copy(table_t.at[f, e, :], row_v, rsem)

        @pl.when(f > 0)
        def _():
            pltpu.make_async_copy(io_v, out.at[f, e, :], ssem).wait()

        pltpu.async_copy(x_f.at[f, :], io_v, isem).wait()
        row_cp.wait()

        def g16(i, carry2):
            s = pl.multiple_of(i * 16, 16)
            idx = plsc.bitcast(io_v[pl.ds(s, 16)], jnp.int32)
            io_v[pl.ds(s, 16)] = plsc.load_gather(row_v, [idx])
            return carry2

        lax.fori_loop(0, B // 16, g16, 0, unroll=16)
        pltpu.async_copy(io_v, out.at[f, e, :], ssem)
        return carry

    lax.fori_loop(0, F, body_f, 0)
    pltpu.make_async_copy(io_v, out.at[F - 1, e, :], ssem).wait()


def kernel(x, embedding):
    table_t = jnp.transpose(embedding, (0, 2, 1))
    x_f = jax.lax.bitcast_convert_type(
        jnp.transpose(x.astype(jnp.int32)), jnp.float32
    )
    out = _sc_lookup(table_t, x_f)
    return jnp.transpose(out, (2, 0, 1))
